# baseline (speedup 1.0000x reference)
import dataclasses
import functools

import jax
import jax.numpy as jnp
from jax.experimental import pallas as pl
from jax.experimental.pallas import tpu as pltpu
from jax.experimental.pallas import tpu_sc as plsc

N_NODES = 10000
N_PAIRS = 160000
GI = 256
HID = 150
HPAD = 256
NC, NS, L = 2, 16, 16
W = 128
NCHUNK = 10
NPAD = 163840
TB = 2048
CH = NPAD // NCHUNK

_SC_PARAMS = pltpu.CompilerParams()
if "needs_layout_passes" in pltpu.CompilerParams.__dataclass_fields__:
    _SC_PARAMS = dataclasses.replace(_SC_PARAMS, needs_layout_passes=False)


def _sc_gather(gi3, ms, mid, aid, db, gb, sb):
    @functools.partial(
        pl.kernel,
        out_type=[
            jax.ShapeDtypeStruct((CH, 128), jnp.int32),
            jax.ShapeDtypeStruct((CH, 128), jnp.int32),
            jax.ShapeDtypeStruct((1, CH), jnp.int32),
            jax.ShapeDtypeStruct((1, CH), jnp.float32),
        ],
        mesh=plsc.VectorSubcoreMesh(core_axis_name="c", subcore_axis_name="s"),
        scratch_types=[
            pltpu.VMEM((N_NODES,), jnp.float32),
            pltpu.SemaphoreType.DMA,
            pltpu.SemaphoreType.DMA,
        ],
        compiler_params=_SC_PARAMS,
    )
    def k(gi_hbm, ms_hbm, mid_hbm, aid_hbm, db_hbm, gb_hbm, sb_hbm,
          ig_hbm, jg_hbm, cidx_hbm, sij_hbm, ms_v, sem1, sem2):
        pltpu.sync_copy(ms_hbm, ms_v)

        def body(mid_v, aid_v, db_v, gb_v, sb_v, ig_o, jg_o, cidx_o, sij_o):
            c1 = pltpu.async_copy(gi_hbm.at[mid_v.at[0]], ig_o, sem1)
            c2 = pltpu.async_copy(gi_hbm.at[aid_v.at[0]], jg_o, sem2)

            @pl.loop(0, W // L)
            def _(j):
                sl = pl.ds(j * L, L)
                cidx_o[0, sl] = db_v[0, sl] * 24 + gb_v[0, sl] * 3 + sb_v[0, sl]
                sij_o[0, sl] = (plsc.load_gather(ms_v, [mid_v[0, sl]])
                                + plsc.load_gather(ms_v, [aid_v[0, sl]]))

            c1.wait()
            c2.wait()

        pltpu.emit_pipeline(
            body,
            grid=(CH // W,),
            in_specs=[pl.BlockSpec((1, W), lambda i: (0, i))] * 5,
            out_specs=[
                pl.BlockSpec((W, 128), lambda i: (i, 0)),
                pl.BlockSpec((W, 128), lambda i: (i, 0)),
                pl.BlockSpec((1, W), lambda i: (0, i)),
                pl.BlockSpec((1, W), lambda i: (0, i)),
            ],
            core_axis_name=("c", "s"),
            dimension_semantics=(pltpu.PARALLEL,),
        )(mid_hbm, aid_hbm, db_hbm, gb_hbm, sb_hbm,
          ig_hbm, jg_hbm, cidx_hbm, sij_hbm)

    return k(gi3, ms, mid, aid, db, gb, sb)


def _pack_body(x_ref, o_ref):
    x = x_ref[...]
    lo = x[:, :128].astype(jnp.bfloat16).astype(jnp.float32)
    hi = x[:, 128:].astype(jnp.bfloat16).astype(jnp.float32)
    lo_i = jax.lax.shift_right_logical(pltpu.bitcast(lo, jnp.int32), 16)
    hi_i = pltpu.bitcast(hi, jnp.int32) & jnp.int32(-65536)
    o_ref[...] = lo_i | hi_i


def _pack_gi(g_i):
    return pl.pallas_call(
        _pack_body,
        grid=(10,),
        in_specs=[pl.BlockSpec((N_NODES // 10, 2 * 128), lambda i: (i, 0))],
        out_specs=pl.BlockSpec((N_NODES // 10, 128), lambda i: (i, 0)),
        out_shape=jax.ShapeDtypeStruct((N_NODES, 128), jnp.int32),
    )(g_i)


def _halves(packed):
    lo = pltpu.bitcast(packed << 16, jnp.float32)
    hi = pltpu.bitcast(packed & jnp.int32(-65536), jnp.float32)
    return lo, hi


def _tc_mlp_body(ig_ref, jg_ref, cidx_ref, sij_ref, w1_ref, cb_ref, w2_ref,
                 b2_ref, w3_ref, b3_ref, out_ref):
    ige_f, igo_f = _halves(ig_ref[...])
    jge_f, jgo_f = _halves(jg_ref[...])
    ige, igo = ige_f.astype(jnp.bfloat16), igo_f.astype(jnp.bfloat16)
    jge, jgo = jge_f.astype(jnp.bfloat16), jgo_f.astype(jnp.bfloat16)
    pe = (ige_f * jge_f).astype(jnp.bfloat16)
    po = (igo_f * jgo_f).astype(jnp.bfloat16)
    w1 = w1_ref[...]
    f32 = jnp.float32
    acc = jnp.dot(ige, w1[0:128], preferred_element_type=f32)
    acc += jnp.dot(igo, w1[128:256], preferred_element_type=f32)
    acc += jnp.dot(jge, w1[256:384], preferred_element_type=f32)
    acc += jnp.dot(jgo, w1[384:512], preferred_element_type=f32)
    acc += jnp.dot(pe, w1[512:640], preferred_element_type=f32)
    acc += jnp.dot(po, w1[640:768], preferred_element_type=f32)
    rows = jax.lax.broadcasted_iota(jnp.int32, (HPAD, TB), 0)
    one_hot_t = (cidx_ref[0] == rows).astype(jnp.bfloat16)
    acc += jax.lax.dot_general(
        one_hot_t, cb_ref[...], (((0,), (0,)), ((), ())),
        preferred_element_type=f32)
    h1 = jnp.maximum(acc, 0.0).astype(jnp.bfloat16)
    h2 = jnp.dot(h1, w2_ref[...], preferred_element_type=f32)
    h2 = jnp.maximum(h2 + b2_ref[...], 0.0)
    pw = jax.lax.dot_general(
        w3_ref[...], h2, (((1,), (1,)), ((), ())),
        preferred_element_type=f32)
    out_ref[...] = (pw + b3_ref[...] + sij_ref[0]).reshape(1, 1, TB)


def _tc_mlp(IG, JG, CIDX, SIJ, W1s, CBp, W2p, b2p, w3p, b3s):
    nblk = CH // TB
    return pl.pallas_call(
        _tc_mlp_body,
        grid=(nblk,),
        in_specs=[
            pl.BlockSpec((TB, 128), lambda i: (i, 0)),
            pl.BlockSpec((TB, 128), lambda i: (i, 0)),
            pl.BlockSpec((1, 1, TB), lambda i: (i, 0, 0)),
            pl.BlockSpec((1, 1, TB), lambda i: (i, 0, 0)),
            pl.BlockSpec((3 * GI, HPAD), lambda i: (0, 0)),
            pl.BlockSpec((HPAD, HPAD), lambda i: (0, 0)),
            pl.BlockSpec((HPAD, HPAD), lambda i: (0, 0)),
            pl.BlockSpec((1, HPAD), lambda i: (0, 0)),
            pl.BlockSpec((1, HPAD), lambda i: (0, 0)),
            pl.BlockSpec((1, 1), lambda i: (0, 0)),
        ],
        out_specs=pl.BlockSpec((1, 1, TB), lambda i: (i, 0, 0)),
        out_shape=jax.ShapeDtypeStruct((nblk, 1, TB), jnp.float32),
    )(IG, JG, CIDX.reshape(nblk, 1, TB), SIJ.reshape(nblk, 1, TB),
      W1s, CBp, W2p, b2p, w3p, b3s)


def kernel(g_i, mention_scores, dist_emb, genre_emb, speaker_emb,
           W1, b1, W2, b2, W3, b3,
           mention_ids, antecedent_ids, distance_bins, genre_ids, speaker_ids):
    gi3 = _pack_gi(g_i)

    cd = dist_emb @ W1[3 * GI:3 * GI + 20]
    cg = genre_emb @ W1[3 * GI + 20:3 * GI + 40]
    cs = speaker_emb @ W1[3 * GI + 40:3 * GI + 60]
    combo = (cd[:, None, None, :] + cg[None, :, None, :]
             + cs[None, None, :, :] + b1).reshape(240, HID)
    CBp = jnp.pad(combo, ((0, HPAD - 240), (0, HPAD - HID))).astype(
        jnp.bfloat16)

    pad = NPAD - N_PAIRS
    mid = jnp.pad(mention_ids.astype(jnp.int32), (0, pad)).reshape(1, NPAD)
    aid = jnp.pad(antecedent_ids.astype(jnp.int32), (0, pad)).reshape(1, NPAD)
    db = jnp.pad(distance_bins.astype(jnp.int32), (0, pad)).reshape(1, NPAD)
    gb = jnp.pad(genre_ids.astype(jnp.int32), (0, pad)).reshape(1, NPAD)
    sb = jnp.pad(speaker_ids.astype(jnp.int32), (0, pad)).reshape(1, NPAD)
    ms = mention_scores.reshape(N_NODES)

    W1s = jnp.pad(W1[:3 * GI], ((0, 0), (0, HPAD - HID))).astype(jnp.bfloat16)
    W2p = jnp.pad(W2, ((0, HPAD - HID), (0, HPAD - HID))).astype(jnp.bfloat16)
    b2p = jnp.pad(b2, (0, HPAD - HID)).reshape(1, HPAD)
    w3p = jnp.pad(W3[:, 0], (0, HPAD - HID)).reshape(1, HPAD)
    b3s = b3.reshape(1, 1)

    outs = []
    for c in range(NCHUNK):
        sl = slice(c * CH, (c + 1) * CH)
        IG, JG, CIDX, SIJ = _sc_gather(
            gi3, ms, mid[:, sl], aid[:, sl], db[:, sl], gb[:, sl], sb[:, sl])
        outs.append(_tc_mlp(IG, JG, CIDX, SIJ,
                            W1s, CBp, W2p, b2p, w3p, b3s).reshape(CH, 1))
    out = jnp.concatenate(outs)
    return out[:N_PAIRS]


# speedup vs baseline: 1.0142x; 1.0142x over previous
import dataclasses
import functools

import jax
import jax.numpy as jnp
from jax.experimental import pallas as pl
from jax.experimental.pallas import tpu as pltpu
from jax.experimental.pallas import tpu_sc as plsc

N_NODES = 10000
N_PAIRS = 160000
GI = 256
HID = 150
HPAD = 256
NC, NS, L = 2, 16, 16
W = 128
NCHUNK = 8
NPAD = 163840
TB = 2048
CH = NPAD // NCHUNK

_SC_PARAMS = pltpu.CompilerParams()
if "needs_layout_passes" in pltpu.CompilerParams.__dataclass_fields__:
    _SC_PARAMS = dataclasses.replace(_SC_PARAMS, needs_layout_passes=False)


def _sc_gather(gi3, ms, mid, aid, db, gb, sb, chunk):
    off = chunk * (CH // W)

    @functools.partial(
        pl.kernel,
        out_type=[
            jax.ShapeDtypeStruct((CH, 128), jnp.int32),
            jax.ShapeDtypeStruct((CH, 128), jnp.int32),
            jax.ShapeDtypeStruct((1, CH), jnp.int32),
            jax.ShapeDtypeStruct((1, CH), jnp.float32),
        ],
        mesh=plsc.VectorSubcoreMesh(core_axis_name="c", subcore_axis_name="s"),
        scratch_types=[
            pltpu.VMEM((N_NODES,), jnp.float32),
            pltpu.SemaphoreType.DMA,
            pltpu.SemaphoreType.DMA,
        ],
        compiler_params=_SC_PARAMS,
    )
    def k(gi_hbm, ms_hbm, mid_hbm, aid_hbm, db_hbm, gb_hbm, sb_hbm,
          ig_hbm, jg_hbm, cidx_hbm, sij_hbm, ms_v, sem1, sem2):
        pltpu.sync_copy(ms_hbm, ms_v)

        def body(mid_v, aid_v, db_v, gb_v, sb_v, ig_o, jg_o, cidx_o, sij_o):
            c1 = pltpu.async_copy(gi_hbm.at[mid_v.at[0]], ig_o, sem1)
            c2 = pltpu.async_copy(gi_hbm.at[aid_v.at[0]], jg_o, sem2)

            @pl.loop(0, W // L)
            def _(j):
                sl = pl.ds(j * L, L)
                cidx_o[0, sl] = db_v[0, sl] * 24 + gb_v[0, sl] * 3 + sb_v[0, sl]
                sij_o[0, sl] = (plsc.load_gather(ms_v, [mid_v[0, sl]])
                                + plsc.load_gather(ms_v, [aid_v[0, sl]]))

            c1.wait()
            c2.wait()

        pltpu.emit_pipeline(
            body,
            grid=(CH // W,),
            in_specs=[pl.BlockSpec((1, W), lambda i: (0, i + off))] * 5,
            out_specs=[
                pl.BlockSpec((W, 128), lambda i: (i, 0)),
                pl.BlockSpec((W, 128), lambda i: (i, 0)),
                pl.BlockSpec((1, W), lambda i: (0, i)),
                pl.BlockSpec((1, W), lambda i: (0, i)),
            ],
            core_axis_name=("c", "s"),
            dimension_semantics=(pltpu.PARALLEL,),
        )(mid_hbm, aid_hbm, db_hbm, gb_hbm, sb_hbm,
          ig_hbm, jg_hbm, cidx_hbm, sij_hbm)

    return k(gi3, ms, mid, aid, db, gb, sb)


def _pack_body(x_ref, o_ref):
    x = x_ref[...]
    lo = x[:, :128].astype(jnp.bfloat16).astype(jnp.float32)
    hi = x[:, 128:].astype(jnp.bfloat16).astype(jnp.float32)
    lo_i = jax.lax.shift_right_logical(pltpu.bitcast(lo, jnp.int32), 16)
    hi_i = pltpu.bitcast(hi, jnp.int32) & jnp.int32(-65536)
    o_ref[...] = lo_i | hi_i


def _pack_gi(g_i):
    return pl.pallas_call(
        _pack_body,
        grid=(10,),
        in_specs=[pl.BlockSpec((N_NODES // 10, 2 * 128), lambda i: (i, 0))],
        out_specs=pl.BlockSpec((N_NODES // 10, 128), lambda i: (i, 0)),
        out_shape=jax.ShapeDtypeStruct((N_NODES, 128), jnp.int32),
    )(g_i)


def _halves(packed):
    lo = pltpu.bitcast(packed << 16, jnp.float32)
    hi = pltpu.bitcast(packed & jnp.int32(-65536), jnp.float32)
    return lo, hi


def _tc_mlp_body(ig_ref, jg_ref, cidx_ref, sij_ref, w1_ref, cb_ref, w2_ref,
                 b2_ref, w3_ref, b3_ref, out_ref):
    ige_f, igo_f = _halves(ig_ref[...])
    jge_f, jgo_f = _halves(jg_ref[...])
    ige, igo = ige_f.astype(jnp.bfloat16), igo_f.astype(jnp.bfloat16)
    jge, jgo = jge_f.astype(jnp.bfloat16), jgo_f.astype(jnp.bfloat16)
    pe = (ige_f * jge_f).astype(jnp.bfloat16)
    po = (igo_f * jgo_f).astype(jnp.bfloat16)
    w1 = w1_ref[...]
    f32 = jnp.float32
    acc = jnp.dot(ige, w1[0:128], preferred_element_type=f32)
    acc += jnp.dot(igo, w1[128:256], preferred_element_type=f32)
    acc += jnp.dot(jge, w1[256:384], preferred_element_type=f32)
    acc += jnp.dot(jgo, w1[384:512], preferred_element_type=f32)
    acc += jnp.dot(pe, w1[512:640], preferred_element_type=f32)
    acc += jnp.dot(po, w1[640:768], preferred_element_type=f32)
    rows = jax.lax.broadcasted_iota(jnp.int32, (HPAD, TB), 0)
    one_hot_t = (cidx_ref[0] == rows).astype(jnp.bfloat16)
    acc += jax.lax.dot_general(
        one_hot_t, cb_ref[...], (((0,), (0,)), ((), ())),
        preferred_element_type=f32)
    h1 = jnp.maximum(acc, 0.0).astype(jnp.bfloat16)
    h2 = jnp.dot(h1, w2_ref[...], preferred_element_type=f32)
    h2 = jnp.maximum(h2 + b2_ref[...], 0.0)
    pw = jax.lax.dot_general(
        w3_ref[...], h2, (((1,), (1,)), ((), ())),
        preferred_element_type=f32)
    out_ref[...] = (pw + b3_ref[...] + sij_ref[0]).reshape(1, 1, TB)


def _tc_mlp(IG, JG, CIDX, SIJ, W1s, CBp, W2p, b2p, w3p, b3s):
    nblk = CH // TB
    return pl.pallas_call(
        _tc_mlp_body,
        grid=(nblk,),
        in_specs=[
            pl.BlockSpec((TB, 128), lambda i: (i, 0)),
            pl.BlockSpec((TB, 128), lambda i: (i, 0)),
            pl.BlockSpec((1, 1, TB), lambda i: (i, 0, 0)),
            pl.BlockSpec((1, 1, TB), lambda i: (i, 0, 0)),
            pl.BlockSpec((3 * GI, HPAD), lambda i: (0, 0)),
            pl.BlockSpec((HPAD, HPAD), lambda i: (0, 0)),
            pl.BlockSpec((HPAD, HPAD), lambda i: (0, 0)),
            pl.BlockSpec((1, HPAD), lambda i: (0, 0)),
            pl.BlockSpec((1, HPAD), lambda i: (0, 0)),
            pl.BlockSpec((1, 1), lambda i: (0, 0)),
        ],
        out_specs=pl.BlockSpec((1, 1, TB), lambda i: (i, 0, 0)),
        out_shape=jax.ShapeDtypeStruct((nblk, 1, TB), jnp.float32),
    )(IG, JG, CIDX.reshape(nblk, 1, TB), SIJ.reshape(nblk, 1, TB),
      W1s, CBp, W2p, b2p, w3p, b3s)


def kernel(g_i, mention_scores, dist_emb, genre_emb, speaker_emb,
           W1, b1, W2, b2, W3, b3,
           mention_ids, antecedent_ids, distance_bins, genre_ids, speaker_ids):
    gi3 = _pack_gi(g_i)

    cd = dist_emb @ W1[3 * GI:3 * GI + 20]
    cg = genre_emb @ W1[3 * GI + 20:3 * GI + 40]
    cs = speaker_emb @ W1[3 * GI + 40:3 * GI + 60]
    combo = (cd[:, None, None, :] + cg[None, :, None, :]
             + cs[None, None, :, :] + b1).reshape(240, HID)
    CBp = jnp.pad(combo, ((0, HPAD - 240), (0, HPAD - HID))).astype(
        jnp.bfloat16)

    pad = NPAD - N_PAIRS
    mid = jnp.pad(mention_ids.astype(jnp.int32), (0, pad)).reshape(1, NPAD)
    aid = jnp.pad(antecedent_ids.astype(jnp.int32), (0, pad)).reshape(1, NPAD)
    db = jnp.pad(distance_bins.astype(jnp.int32), (0, pad)).reshape(1, NPAD)
    gb = jnp.pad(genre_ids.astype(jnp.int32), (0, pad)).reshape(1, NPAD)
    sb = jnp.pad(speaker_ids.astype(jnp.int32), (0, pad)).reshape(1, NPAD)
    ms = mention_scores.reshape(N_NODES)

    W1s = jnp.pad(W1[:3 * GI], ((0, 0), (0, HPAD - HID))).astype(jnp.bfloat16)
    W2p = jnp.pad(W2, ((0, HPAD - HID), (0, HPAD - HID))).astype(jnp.bfloat16)
    b2p = jnp.pad(b2, (0, HPAD - HID)).reshape(1, HPAD)
    w3p = jnp.pad(W3[:, 0], (0, HPAD - HID)).reshape(1, HPAD)
    b3s = b3.reshape(1, 1)

    outs = []
    for c in range(NCHUNK):
        IG, JG, CIDX, SIJ = _sc_gather(gi3, ms, mid, aid, db, gb, sb, c)
        outs.append(_tc_mlp(IG, JG, CIDX, SIJ,
                            W1s, CBp, W2p, b2p, w3p, b3s).reshape(CH, 1))
    out = jnp.concatenate(outs)
    return out[:N_PAIRS]
